# baseline (speedup 1.0000x reference)
import functools

import jax
import jax.numpy as jnp
from jax import lax
from jax.experimental import pallas as pl
from jax.experimental.pallas import tpu as pltpu
from jax.experimental.pallas import tpu_sc as plsc

N_TOKENS = 4096
N_IN = 1024
N_OUT = 1024
N_EXPERTS = 8
TILE_M = 256
P_MAX = N_TOKENS * 2 + N_EXPERTS * TILE_M
NT = P_MAX // TILE_M
NW = 32
TOK_PER_W = N_TOKENS // NW
SUB = 32
NSUB = TOK_PER_W // SUB


def _cumsum_rows(x):
    n = x.shape[0]
    sh = 1
    while sh < n:
        pad = jnp.zeros((sh, x.shape[1]), x.dtype)
        x = x + jnp.concatenate([pad, x[:-sh, :]], axis=0)
        sh *= 2
    return x


def _routing_body(x_ref, wsel_ref, bsel_ref,
                  pos0_ref, pos1_ref, g0_ref, g1_ref, te_ref, aux_ref):
    xb = x_ref[...].astype(jnp.bfloat16)
    wselb = wsel_ref[...].astype(jnp.bfloat16)
    logits = (
        jnp.dot(xb, wselb, preferred_element_type=jnp.float32) + bsel_ref[...]
    )
    m = jnp.max(logits, axis=-1, keepdims=True)
    e = jnp.exp(logits - m)
    s = jnp.sum(e, axis=-1, keepdims=True)
    p = e / s

    row_sums = jnp.sum(p, axis=-1)
    mean = jnp.mean(row_sums)
    var = jnp.mean((row_sums - mean) ** 2)
    aux_ref[...] = (var / (mean * mean + 1e-10)).reshape(1, 1)

    iota = lax.broadcasted_iota(jnp.int32, p.shape, 1)
    max1 = jnp.max(p, axis=-1, keepdims=True)
    i1 = jnp.min(jnp.where(p == max1, iota, N_EXPERTS), axis=-1, keepdims=True)
    m1 = iota == i1
    p2 = jnp.where(m1, -1.0, p)
    max2 = jnp.max(p2, axis=-1, keepdims=True)
    i2 = jnp.min(jnp.where(p2 == max2, iota, N_EXPERTS), axis=-1, keepdims=True)
    m2 = iota == i2

    g0_ref[...] = jnp.broadcast_to(max1, (N_TOKENS, 16))
    g1_ref[...] = jnp.broadcast_to(max2, (N_TOKENS, 16))

    mask = (m1 | m2).astype(jnp.int32)
    incl = _cumsum_rows(mask)
    cnt = incl[N_TOKENS - 1:N_TOKENS, :]
    cntp = ((cnt + (TILE_M - 1)) // TILE_M) * TILE_M
    base = jnp.zeros_like(cntp)
    sh = 1
    acc = cntp
    while sh < N_EXPERTS:
        pad = jnp.zeros((1, sh), jnp.int32)
        acc = acc + jnp.concatenate([pad, acc[:, :-sh]], axis=1)
        sh *= 2
    base = acc - cntp

    rank = incl - mask
    pos_e = base + rank
    m1i = m1.astype(jnp.int32)
    m2i = m2.astype(jnp.int32)
    pos0_ref[...] = jnp.sum(m1i * pos_e, axis=-1, keepdims=True)
    pos1_ref[...] = jnp.sum(m2i * pos_e, axis=-1, keepdims=True)

    ends = base + cntp
    j = lax.broadcasted_iota(jnp.int32, (64, 1), 0) * TILE_M
    te = jnp.sum((j >= ends).astype(jnp.int32), axis=-1, keepdims=True)
    te_ref[...] = jnp.minimum(te, N_EXPERTS - 1)


def _grouped_body(te_ref, xg_ref, w_ref, b_ref, yg_ref):
    xgb = xg_ref[...].astype(jnp.bfloat16)
    wb = w_ref[0].astype(jnp.bfloat16)
    y = jnp.dot(xgb, wb, preferred_element_type=jnp.float32) + b_ref[0]
    yg_ref[...] = jnp.maximum(y, 0.0)


def _make_dispatch():
    mesh = plsc.VectorSubcoreMesh(core_axis_name="c", subcore_axis_name="s")

    @functools.partial(
        pl.kernel,
        mesh=mesh,
        out_type=jax.ShapeDtypeStruct((P_MAX, N_IN), jnp.float32),
        scratch_types=[
            pltpu.VMEM((SUB,), jnp.int32),
            pltpu.VMEM((SUB,), jnp.int32),
            pltpu.VMEM((SUB, N_IN), jnp.float32),
            pltpu.SemaphoreType.DMA,
        ],
    )
    def dispatch(x_hbm, pos0_hbm, pos1_hbm, xg_hbm, idx0_v, idx1_v, rows_v,
                 sem):
        wid = lax.axis_index("s") * 2 + lax.axis_index("c")
        for c in range(NSUB):
            tok0 = wid * TOK_PER_W + c * SUB
            pltpu.sync_copy(pos0_hbm.at[wid, c], idx0_v)
            pltpu.sync_copy(pos1_hbm.at[wid, c], idx1_v)
            pltpu.sync_copy(x_hbm.at[pl.ds(tok0, SUB)], rows_v)
            pltpu.async_copy(rows_v, xg_hbm.at[idx0_v], sem).wait()
            pltpu.async_copy(rows_v, xg_hbm.at[idx1_v], sem).wait()

    return dispatch


def _make_combine():
    mesh = plsc.VectorSubcoreMesh(core_axis_name="c", subcore_axis_name="s")

    @functools.partial(
        pl.kernel,
        mesh=mesh,
        out_type=jax.ShapeDtypeStruct((N_TOKENS, N_OUT), jnp.float32),
        scratch_types=[
            pltpu.VMEM((SUB,), jnp.int32),
            pltpu.VMEM((SUB,), jnp.int32),
            pltpu.VMEM((SUB, N_OUT), jnp.float32),
            pltpu.VMEM((SUB, N_OUT), jnp.float32),
            pltpu.VMEM((SUB, 16), jnp.float32),
            pltpu.VMEM((SUB, 16), jnp.float32),
            pltpu.VMEM((SUB, N_OUT), jnp.float32),
            pltpu.SemaphoreType.DMA,
        ],
    )
    def combine(yg_hbm, pos0_hbm, pos1_hbm, g0_hbm, g1_hbm, out_hbm,
                idx0_v, idx1_v, r0_v, r1_v, gv0_v, gv1_v, out_v, sem):
        wid = lax.axis_index("s") * 2 + lax.axis_index("c")
        for c in range(NSUB):
            tok0 = wid * TOK_PER_W + c * SUB
            pltpu.sync_copy(pos0_hbm.at[wid, c], idx0_v)
            pltpu.sync_copy(pos1_hbm.at[wid, c], idx1_v)
            pltpu.sync_copy(g0_hbm.at[pl.ds(tok0, SUB)], gv0_v)
            pltpu.sync_copy(g1_hbm.at[pl.ds(tok0, SUB)], gv1_v)
            cp0 = pltpu.async_copy(yg_hbm.at[idx0_v], r0_v, sem)
            cp1 = pltpu.async_copy(yg_hbm.at[idx1_v], r1_v, sem)
            cp0.wait()
            cp1.wait()

            def token_body(t, _):
                gv0 = gv0_v[t]
                gv1 = gv1_v[t]

                def vec_body(j, _):
                    v = gv0 * r0_v[t, pl.ds(j * 16, 16)]
                    v = v + gv1 * r1_v[t, pl.ds(j * 16, 16)]
                    out_v[t, pl.ds(j * 16, 16)] = v
                    return 0

                return lax.fori_loop(0, N_OUT // 16, vec_body, 0)

            lax.fori_loop(0, SUB, token_body, 0)
            pltpu.sync_copy(out_v, out_hbm.at[pl.ds(tok0, SUB)])

    return combine


@functools.partial(jax.jit, static_argnames=())
def kernel(inputs, Wsel, bsel, W, b):
    pos0, pos1, g0, g1, te, aux = pl.pallas_call(
        _routing_body,
        out_shape=(
            jax.ShapeDtypeStruct((N_TOKENS, 1), jnp.int32),
            jax.ShapeDtypeStruct((N_TOKENS, 1), jnp.int32),
            jax.ShapeDtypeStruct((N_TOKENS, 16), jnp.float32),
            jax.ShapeDtypeStruct((N_TOKENS, 16), jnp.float32),
            jax.ShapeDtypeStruct((64, 1), jnp.int32),
            jax.ShapeDtypeStruct((1, 1), jnp.float32),
        ),
    )(inputs, Wsel, bsel.reshape(1, N_EXPERTS))

    pos0w = pos0.reshape(NW, NSUB, SUB)
    pos1w = pos1.reshape(NW, NSUB, SUB)

    xg = _make_dispatch()(inputs, pos0w, pos1w)

    yg = pl.pallas_call(
        _grouped_body,
        grid_spec=pltpu.PrefetchScalarGridSpec(
            num_scalar_prefetch=1,
            grid=(NT,),
            in_specs=[
                pl.BlockSpec((TILE_M, N_IN), lambda i, te: (i, 0)),
                pl.BlockSpec((1, N_IN, N_OUT), lambda i, te: (te[i], 0, 0)),
                pl.BlockSpec((1, 1, N_OUT), lambda i, te: (te[i], 0, 0)),
            ],
            out_specs=pl.BlockSpec((TILE_M, N_OUT), lambda i, te: (i, 0)),
        ),
        out_shape=jax.ShapeDtypeStruct((P_MAX, N_OUT), jnp.float32),
        compiler_params=pltpu.CompilerParams(
            dimension_semantics=("arbitrary",),
        ),
    )(te.reshape(64), xg, W, b.reshape(N_EXPERTS, 1, N_OUT))

    out = _make_combine()(yg, pos0w, pos1w, g0, g1)
    return (out, aux.reshape(()))
